# baseline (device time: 105475 ns/iter reference)
import jax
import jax.numpy as jnp
from jax import lax
from jax.experimental import pallas as pl
from jax.experimental.pallas import tpu as pltpu

N_DEV = 4
EPS = 1e-5
BM = 1024
NBLK = 8
NCACHE = 2
ROWS = BM // 128

PH2_ORDER = [0, 1, 6, 7, 2, 3, 4, 5]
PH2_SLOT = {6: 0, 7: 1, 2: 0, 3: 1, 4: 0, 5: 1}
PH2_REFETCH_AFTER = {6: 2, 7: 3, 2: 4, 3: 5}


def _body(global_n, x_hbm, g_ref, out_hbm, xc, xs, osb, comm,
          c_sems, in_sems, o_sems, send_sems, recv_sems):
    my = lax.axis_index("i")

    r128 = lax.broadcasted_iota(jnp.int32, (BM, 128), 0) % 128
    l128 = lax.broadcasted_iota(jnp.int32, (BM, 128), 1)
    q_mask = r128 == l128
    t_idx = lax.broadcasted_iota(jnp.int32, (ROWS, BM), 0)
    rr_idx = lax.broadcasted_iota(jnp.int32, (ROWS, BM), 1)
    m1 = jnp.where(rr_idx // 128 == t_idx, 1.0, 0.0).astype(jnp.float32)
    p0 = jnp.where(
        lax.broadcasted_iota(jnp.int32, (BM, ROWS), 1)
        == lax.broadcasted_iota(jnp.int32, (BM, ROWS), 0) // 128,
        1.0, 0.0).astype(jnp.float32)

    def fill(b, slot_ref, sem):
        cp = pltpu.make_async_copy(
            x_hbm.at[pl.ds(b * BM, BM), :], slot_ref, sem)
        cp.start()
        return cp

    def ph1_slot(b):
        if b < NCACHE:
            return xc.at[b], c_sems.at[b]
        return xs.at[(b - NCACHE) % 2], in_sems.at[(b - NCACHE) % 2]

    fills = {b: fill(b, *ph1_slot(b)) for b in range(NCACHE + 2)}
    for b in range(NBLK):
        fills[b].wait()
        xb = xc[b] if b < NCACHE else xs[(b - NCACHE) % 2]
        s = jnp.sum(xb * xb, axis=1, keepdims=True)
        w = jnp.where(q_mask, s, 0.0)
        c = jax.lax.dot(m1, w, preferred_element_type=jnp.float32)
        comm[0, pl.ds(b * ROWS, ROWS), :] = c
        if NCACHE <= b < NBLK - 2:
            fills[b + 2] = fill(b + 2, *ph1_slot(b + 2))

    barrier = pltpu.get_barrier_semaphore()
    for j in range(1, N_DEV):
        pl.semaphore_signal(
            barrier, inc=1, device_id=(lax.rem(my + j, N_DEV),),
            device_id_type=pl.DeviceIdType.MESH,
        )
    pl.semaphore_wait(barrier, N_DEV - 1)

    rdmas = []
    for j in range(1, N_DEV):
        rdma = pltpu.make_async_remote_copy(
            src_ref=comm.at[0],
            dst_ref=comm.at[N_DEV - j],
            send_sem=send_sems.at[j - 1],
            recv_sem=recv_sems.at[N_DEV - 1 - j],
            device_id=(lax.rem(my + j, N_DEV),),
            device_id_type=pl.DeviceIdType.MESH,
        )
        rdma.start()
        rdmas.append(rdma)
    for rdma in rdmas:
        rdma.wait()

    total = (comm[0, :, :] + comm[1, :, :]
             + comm[2, :, :] + comm[3, :, :])
    inv2 = lax.rsqrt(total * (1.0 / global_n) + EPS)

    gam = g_ref[...]

    def inv_col(b):
        t = jax.lax.dot(p0, inv2[b * ROWS:(b + 1) * ROWS, :],
                        preferred_element_type=jnp.float32)
        return jnp.sum(jnp.where(q_mask, t, 0.0), axis=1, keepdims=True)

    refetch = {}
    out_cps = []
    for k, b in enumerate(PH2_ORDER):
        if b in refetch:
            refetch[b].wait()
        xv = xc[b] if b < NCACHE else xs[PH2_SLOT[b]]
        if k >= 2:
            out_cps[k - 2].wait()
        osb[k % 2, :, :] = xv * inv_col(b) * gam
        cp = pltpu.make_async_copy(
            osb.at[k % 2], out_hbm.at[pl.ds(b * BM, BM), :],
            o_sems.at[k % 2])
        cp.start()
        out_cps.append(cp)
        v = PH2_REFETCH_AFTER.get(b)
        if v is not None:
            refetch[v] = fill(v, xs.at[PH2_SLOT[v]], in_sems.at[PH2_SLOT[v]])
    out_cps[-2].wait()
    out_cps[-1].wait()


def kernel(x, gamma):
    m, n_loc = x.shape
    global_n = n_loc * N_DEV
    g2 = gamma.reshape(1, n_loc)

    return pl.pallas_call(
        lambda *refs: _body(global_n, *refs),
        in_specs=[
            pl.BlockSpec(memory_space=pl.ANY),
            pl.BlockSpec(memory_space=pltpu.VMEM),
        ],
        out_specs=pl.BlockSpec(memory_space=pl.ANY),
        out_shape=jax.ShapeDtypeStruct((m, n_loc), jnp.float32),
        scratch_shapes=[
            pltpu.VMEM((NCACHE, BM, n_loc), jnp.float32),
            pltpu.VMEM((2, BM, n_loc), jnp.float32),
            pltpu.VMEM((2, BM, n_loc), jnp.float32),
            pltpu.VMEM((N_DEV, m // 128, 128), jnp.float32),
            pltpu.SemaphoreType.DMA((NCACHE,)),
            pltpu.SemaphoreType.DMA((2,)),
            pltpu.SemaphoreType.DMA((2,)),
            pltpu.SemaphoreType.DMA((N_DEV - 1,)),
            pltpu.SemaphoreType.DMA((N_DEV - 1,)),
        ],
        compiler_params=pltpu.CompilerParams(
            collective_id=0, vmem_limit_bytes=64 * 1024 * 1024,
        ),
    )(x, g2)


# device time: 104297 ns/iter; 1.0113x vs baseline; 1.0113x over previous
import jax
import jax.numpy as jnp
from jax import lax
from jax.experimental import pallas as pl
from jax.experimental.pallas import tpu as pltpu

N_DEV = 4
EPS = 1e-5
BM = 1024
NBLK = 8
ROWS = BM // 128

PH1_BUF = {0: "c0", 1: "c1", 2: "s0", 3: "s1", 4: "s0", 5: "s1",
           6: "s0", 7: "s1"}
PH2_ORDER = [0, 1, 6, 7, 2, 3, 4, 5]
PH2_BUF = {0: "c0", 1: "c1", 6: "s0", 7: "s1", 2: "s0", 3: "s1",
           4: "s0", 5: "s1"}
PH2_REFETCH_AFTER = {6: 2, 7: 3, 2: 4, 3: 5}


def _body(global_n, x_hbm, g_ref, out_hbm, c0, c1, s0, s1, o0, o1, comm,
          in_sems, o_sems, send_sems, recv_sems):
    my = lax.axis_index("i")
    bufs = {"c0": c0, "c1": c1, "s0": s0, "s1": s1}
    sem_ix = {"c0": 0, "c1": 1, "s0": 2, "s1": 3}

    r128 = lax.broadcasted_iota(jnp.int32, (BM, 128), 0) % 128
    l128 = lax.broadcasted_iota(jnp.int32, (BM, 128), 1)
    q_mask = r128 == l128
    t_idx = lax.broadcasted_iota(jnp.int32, (ROWS, BM), 0)
    rr_idx = lax.broadcasted_iota(jnp.int32, (ROWS, BM), 1)
    m1 = jnp.where(rr_idx // 128 == t_idx, 1.0, 0.0).astype(jnp.float32)
    p0 = jnp.where(
        lax.broadcasted_iota(jnp.int32, (BM, ROWS), 1)
        == lax.broadcasted_iota(jnp.int32, (BM, ROWS), 0) // 128,
        1.0, 0.0).astype(jnp.float32)

    def fill(b, name):
        cp = pltpu.make_async_copy(
            x_hbm.at[pl.ds(b * BM, BM), :], bufs[name],
            in_sems.at[sem_ix[name]])
        cp.start()
        return cp

    fills = {b: fill(b, PH1_BUF[b]) for b in range(4)}
    for b in range(NBLK):
        fills[b].wait()
        xb = bufs[PH1_BUF[b]][...]
        s = jnp.sum(xb * xb, axis=1, keepdims=True)
        w = jnp.where(q_mask, s, 0.0)
        c = jax.lax.dot(m1, w, preferred_element_type=jnp.float32)
        comm[0, pl.ds(b * ROWS, ROWS), :] = c
        if 2 <= b < NBLK - 2:
            fills[b + 2] = fill(b + 2, PH1_BUF[b + 2])

    barrier = pltpu.get_barrier_semaphore()
    for j in range(1, N_DEV):
        pl.semaphore_signal(
            barrier, inc=1, device_id=(lax.rem(my + j, N_DEV),),
            device_id_type=pl.DeviceIdType.MESH,
        )
    pl.semaphore_wait(barrier, N_DEV - 1)

    rdmas = []
    for j in range(1, N_DEV):
        rdma = pltpu.make_async_remote_copy(
            src_ref=comm.at[0],
            dst_ref=comm.at[N_DEV - j],
            send_sem=send_sems.at[j - 1],
            recv_sem=recv_sems.at[N_DEV - 1 - j],
            device_id=(lax.rem(my + j, N_DEV),),
            device_id_type=pl.DeviceIdType.MESH,
        )
        rdma.start()
        rdmas.append(rdma)
    for rdma in rdmas:
        rdma.wait()

    total = (comm[0, :, :] + comm[1, :, :]
             + comm[2, :, :] + comm[3, :, :])
    inv2 = lax.rsqrt(total * (1.0 / global_n) + EPS)

    gam = g_ref[...]
    obufs = [o0, o1]

    def inv_col(b):
        t = jax.lax.dot(p0, inv2[b * ROWS:(b + 1) * ROWS, :],
                        preferred_element_type=jnp.float32)
        return jnp.sum(jnp.where(q_mask, t, 0.0), axis=1, keepdims=True)

    refetch = {}
    out_cps = []
    for k, b in enumerate(PH2_ORDER):
        if b in refetch:
            refetch[b].wait()
        if k >= 2:
            out_cps[k - 2].wait()
        ob = obufs[k % 2]
        ob[...] = bufs[PH2_BUF[b]][...] * inv_col(b) * gam
        cp = pltpu.make_async_copy(
            ob, out_hbm.at[pl.ds(b * BM, BM), :], o_sems.at[k % 2])
        cp.start()
        out_cps.append(cp)
        v = PH2_REFETCH_AFTER.get(b)
        if v is not None:
            refetch[v] = fill(v, PH2_BUF[v])
    out_cps[-2].wait()
    out_cps[-1].wait()


def kernel(x, gamma):
    m, n_loc = x.shape
    global_n = n_loc * N_DEV
    g2 = gamma.reshape(1, n_loc)

    blk = pltpu.VMEM((BM, n_loc), jnp.float32)
    return pl.pallas_call(
        lambda *refs: _body(global_n, *refs),
        in_specs=[
            pl.BlockSpec(memory_space=pl.ANY),
            pl.BlockSpec(memory_space=pltpu.VMEM),
        ],
        out_specs=pl.BlockSpec(memory_space=pl.ANY),
        out_shape=jax.ShapeDtypeStruct((m, n_loc), jnp.float32),
        scratch_shapes=[
            blk, blk,
            blk, blk,
            blk, blk,
            pltpu.VMEM((N_DEV, m // 128, 128), jnp.float32),
            pltpu.SemaphoreType.DMA((4,)),
            pltpu.SemaphoreType.DMA((2,)),
            pltpu.SemaphoreType.DMA((N_DEV - 1,)),
            pltpu.SemaphoreType.DMA((N_DEV - 1,)),
        ],
        compiler_params=pltpu.CompilerParams(
            collective_id=0, vmem_limit_bytes=64 * 1024 * 1024,
        ),
    )(x, g2)


# device time: 104260 ns/iter; 1.0117x vs baseline; 1.0004x over previous
import jax
import jax.numpy as jnp
from jax import lax
from jax.experimental import pallas as pl
from jax.experimental.pallas import tpu as pltpu

N_DEV = 4
EPS = 1e-5
BM = 1024
NBLK = 8
ROWS = BM // 128

PH1_BUF = {0: "c0", 1: "c1", 2: "s0", 3: "s1", 4: "s0", 5: "s1",
           6: "s0", 7: "s1"}
PH2_ORDER = [0, 1, 6, 7, 2, 3, 4, 5]
PH2_BUF = {0: "c0", 1: "c1", 6: "s0", 7: "s1", 2: "s0", 3: "s1",
           4: "s0", 5: "s1"}
PH2_REFETCH_AFTER = {6: 2, 7: 3, 2: 4, 3: 5}


def _body(global_n, x_hbm, g_ref, out_hbm, c0, c1, s0, s1, o0, o1, comm,
          in_sems, o_sems, send_sems, recv_sems):
    my = lax.axis_index("i")
    bufs = {"c0": c0, "c1": c1, "s0": s0, "s1": s1}
    sem_ix = {"c0": 0, "c1": 1, "s0": 2, "s1": 3}

    r128 = lax.broadcasted_iota(jnp.int32, (BM, 128), 0) % 128
    l128 = lax.broadcasted_iota(jnp.int32, (BM, 128), 1)
    q_mask = r128 == l128
    t_idx = lax.broadcasted_iota(jnp.int32, (ROWS, BM), 0)
    rr_idx = lax.broadcasted_iota(jnp.int32, (ROWS, BM), 1)
    m1 = jnp.where(rr_idx // 128 == t_idx, 1.0, 0.0).astype(jnp.float32)
    p0 = jnp.where(
        lax.broadcasted_iota(jnp.int32, (BM, ROWS), 1)
        == lax.broadcasted_iota(jnp.int32, (BM, ROWS), 0) // 128,
        1.0, 0.0).astype(jnp.float32)

    def fill(b, name):
        cp = pltpu.make_async_copy(
            x_hbm.at[pl.ds(b * BM, BM), :], bufs[name],
            in_sems.at[sem_ix[name]])
        cp.start()
        return cp

    fills = {b: fill(b, PH1_BUF[b]) for b in range(4)}
    for b in range(NBLK):
        with jax.named_scope(f"ph1#b={b}"):
            fills[b].wait()
            xb = bufs[PH1_BUF[b]][...]
            s = jnp.sum(xb * xb, axis=1, keepdims=True)
            w = jnp.where(q_mask, s, 0.0)
            c = jax.lax.dot(m1, w, preferred_element_type=jnp.float32)
            comm[0, pl.ds(b * ROWS, ROWS), :] = c
            if 2 <= b < NBLK - 2:
                fills[b + 2] = fill(b + 2, PH1_BUF[b + 2])

    scope_comm = jax.named_scope("comm")
    scope_comm.__enter__()
    barrier = pltpu.get_barrier_semaphore()
    for j in range(1, N_DEV):
        pl.semaphore_signal(
            barrier, inc=1, device_id=(lax.rem(my + j, N_DEV),),
            device_id_type=pl.DeviceIdType.MESH,
        )
    pl.semaphore_wait(barrier, N_DEV - 1)

    rdmas = []
    for j in range(1, N_DEV):
        rdma = pltpu.make_async_remote_copy(
            src_ref=comm.at[0],
            dst_ref=comm.at[N_DEV - j],
            send_sem=send_sems.at[j - 1],
            recv_sem=recv_sems.at[N_DEV - 1 - j],
            device_id=(lax.rem(my + j, N_DEV),),
            device_id_type=pl.DeviceIdType.MESH,
        )
        rdma.start()
        rdmas.append(rdma)
    for rdma in rdmas:
        rdma.wait()

    total = (comm[0, :, :] + comm[1, :, :]
             + comm[2, :, :] + comm[3, :, :])
    inv2 = lax.rsqrt(total * (1.0 / global_n) + EPS)
    scope_comm.__exit__(None, None, None)

    gam = g_ref[...]
    obufs = [o0, o1]

    def inv_col(b):
        t = jax.lax.dot(p0, inv2[b * ROWS:(b + 1) * ROWS, :],
                        preferred_element_type=jnp.float32)
        return jnp.sum(jnp.where(q_mask, t, 0.0), axis=1, keepdims=True)

    refetch = {}
    out_cps = []
    for k, b in enumerate(PH2_ORDER):
        with jax.named_scope(f"ph2#b={b}"):
            if b in refetch:
                refetch[b].wait()
            if k >= 2:
                out_cps[k - 2].wait()
            ob = obufs[k % 2]
            ob[...] = bufs[PH2_BUF[b]][...] * inv_col(b) * gam
            cp = pltpu.make_async_copy(
                ob, out_hbm.at[pl.ds(b * BM, BM), :], o_sems.at[k % 2])
            cp.start()
            out_cps.append(cp)
            v = PH2_REFETCH_AFTER.get(b)
            if v is not None:
                refetch[v] = fill(v, PH2_BUF[v])
    out_cps[-2].wait()
    out_cps[-1].wait()


def kernel(x, gamma):
    m, n_loc = x.shape
    global_n = n_loc * N_DEV
    g2 = gamma.reshape(1, n_loc)

    blk = pltpu.VMEM((BM, n_loc), jnp.float32)
    return pl.pallas_call(
        lambda *refs: _body(global_n, *refs),
        in_specs=[
            pl.BlockSpec(memory_space=pltpu.MemorySpace.HBM),
            pl.BlockSpec(memory_space=pltpu.VMEM),
        ],
        out_specs=pl.BlockSpec(memory_space=pltpu.MemorySpace.HBM),
        out_shape=jax.ShapeDtypeStruct((m, n_loc), jnp.float32),
        scratch_shapes=[
            blk, blk,
            blk, blk,
            blk, blk,
            pltpu.VMEM((N_DEV, m // 128, 128), jnp.float32),
            pltpu.SemaphoreType.DMA((4,)),
            pltpu.SemaphoreType.DMA((2,)),
            pltpu.SemaphoreType.DMA((N_DEV - 1,)),
            pltpu.SemaphoreType.DMA((N_DEV - 1,)),
        ],
        compiler_params=pltpu.CompilerParams(
            collective_id=0, vmem_limit_bytes=64 * 1024 * 1024,
        ),
    )(x, g2)
